# initial kernel scaffold (unmeasured)
import jax
import jax.numpy as jnp
from jax import lax
from jax.experimental import pallas as pl
from jax.experimental.pallas import tpu as pltpu

N_DEV = 4
B = 16
SEQ = 1024
H = 16
D = 64
F = H * D
SCALE = D ** -0.5


def _expand_hf():
    h = lax.broadcasted_iota(jnp.int32, (H, F), 0)
    f = lax.broadcasted_iota(jnp.int32, (H, F), 1)
    return (f // D == h).astype(jnp.float32)


def _mask_fh():
    f = lax.broadcasted_iota(jnp.int32, (F, H), 0)
    h = lax.broadcasted_iota(jnp.int32, (F, H), 1)
    return (f // D == h).astype(jnp.float32)


def _partial_body(q_ref, k_ref, v_ref, o_ref, m_ref, l_ref):
    qcol = jnp.reshape(q_ref[...], (F, 1))
    w = qcol * _mask_fh()
    k2 = k_ref[0]
    s = jax.lax.dot_general(
        k2, w, (((1,), (0,)), ((), ())),
        preferred_element_type=jnp.float32,
    ) * SCALE
    m = jnp.max(s, axis=0, keepdims=True)
    p = jnp.exp(s - m)
    l = jnp.sum(p, axis=0, keepdims=True)
    pe = jax.lax.dot_general(
        p, _expand_hf(), (((1,), (0,)), ((), ())),
        preferred_element_type=jnp.float32,
    )
    o_ref[...] = jnp.sum(pe * v_ref[0], axis=0, keepdims=True)
    m_ref[...] = m
    l_ref[...] = l


def _partials(q2, k2, v2):
    return pl.pallas_call(
        _partial_body,
        grid=(B,),
        in_specs=[
            pl.BlockSpec((1, F), lambda b: (b, 0)),
            pl.BlockSpec((1, SEQ, F), lambda b: (b, 0, 0)),
            pl.BlockSpec((1, SEQ, F), lambda b: (b, 0, 0)),
        ],
        out_specs=[
            pl.BlockSpec((1, F), lambda b: (b, 0)),
            pl.BlockSpec((1, H), lambda b: (b, 0)),
            pl.BlockSpec((1, H), lambda b: (b, 0)),
        ],
        out_shape=[
            jax.ShapeDtypeStruct((B, F), jnp.float32),
            jax.ShapeDtypeStruct((B, H), jnp.float32),
            jax.ShapeDtypeStruct((B, H), jnp.float32),
        ],
    )(q2, k2, v2)


def _combine_body(o_ref, m_ref, l_ref, out_ref,
                  co, cm, cl, so, sm, sl, ro, rm, rl):
    my = lax.axis_index("i")

    bar = pltpu.get_barrier_semaphore()
    for k in (1, 2, 3):
        pl.semaphore_signal(
            bar, inc=1,
            device_id=((my + k) % N_DEV,),
            device_id_type=pl.DeviceIdType.MESH,
        )
    pl.semaphore_wait(bar, 3)

    co[0] = o_ref[...]
    cm[0] = m_ref[...]
    cl[0] = l_ref[...]

    rdmas = []
    for k in (1, 2, 3):
        tgt = (my + k) % N_DEV
        slot = N_DEV - k
        for comm, ssem, rsem in ((co, so, ro), (cm, sm, rm), (cl, sl, rl)):
            r = pltpu.make_async_remote_copy(
                src_ref=comm.at[0],
                dst_ref=comm.at[slot],
                send_sem=ssem.at[k],
                recv_sem=rsem.at[slot],
                device_id=(tgt,),
                device_id_type=pl.DeviceIdType.MESH,
            )
            r.start()
            rdmas.append(r)
    for r in rdmas:
        r.wait()

    e = _expand_hf()
    ms = [cm[i] for i in range(N_DEV)]
    mg = jnp.maximum(jnp.maximum(ms[0], ms[1]), jnp.maximum(ms[2], ms[3]))
    lg = jnp.zeros((B, H), jnp.float32)
    acc = jnp.zeros((B, F), jnp.float32)
    for i in range(N_DEV):
        wi = jnp.exp(ms[i] - mg)
        lg = lg + wi * cl[i]
        wf = jax.lax.dot_general(
            wi, e, (((1,), (0,)), ((), ())),
            preferred_element_type=jnp.float32,
        )
        acc = acc + wf * co[i]
    lgf = jax.lax.dot_general(
        lg, e, (((1,), (0,)), ((), ())),
        preferred_element_type=jnp.float32,
    )
    out_ref[...] = acc / lgf


def _combine(o_part, m_part, l_part):
    return pl.pallas_call(
        _combine_body,
        in_specs=[
            pl.BlockSpec(memory_space=pltpu.VMEM),
            pl.BlockSpec(memory_space=pltpu.VMEM),
            pl.BlockSpec(memory_space=pltpu.VMEM),
        ],
        out_specs=pl.BlockSpec(memory_space=pltpu.VMEM),
        out_shape=jax.ShapeDtypeStruct((B, F), jnp.float32),
        scratch_shapes=[
            pltpu.VMEM((N_DEV, B, F), jnp.float32),
            pltpu.VMEM((N_DEV, B, H), jnp.float32),
            pltpu.VMEM((N_DEV, B, H), jnp.float32),
            pltpu.SemaphoreType.DMA((N_DEV,)),
            pltpu.SemaphoreType.DMA((N_DEV,)),
            pltpu.SemaphoreType.DMA((N_DEV,)),
            pltpu.SemaphoreType.DMA((N_DEV,)),
            pltpu.SemaphoreType.DMA((N_DEV,)),
            pltpu.SemaphoreType.DMA((N_DEV,)),
        ],
        compiler_params=pltpu.CompilerParams(collective_id=0),
    )(o_part, m_part, l_part)


def kernel(Q, K, V):
    q2 = Q.reshape(B, F)
    k2 = K.reshape(B, SEQ, F)
    v2 = V.reshape(B, SEQ, F)
    o_part, m_part, l_part = _partials(q2, k2, v2)
    out = _combine(o_part, m_part, l_part)
    return out.reshape(B, 1, H, D)


# baseline (device time: 187589 ns/iter reference)
import jax
import jax.numpy as jnp
from jax import lax
from jax.experimental import pallas as pl
from jax.experimental.pallas import tpu as pltpu

N_DEV = 4
B = 16
SEQ = 1024
H = 16
D = 64
F = H * D
SCALE = D ** -0.5


def _expand_hf():
    h = lax.broadcasted_iota(jnp.int32, (H, F), 0)
    f = lax.broadcasted_iota(jnp.int32, (H, F), 1)
    return (f // D == h).astype(jnp.float32)


def _mask_fh():
    f = lax.broadcasted_iota(jnp.int32, (F, H), 0)
    h = lax.broadcasted_iota(jnp.int32, (F, H), 1)
    return (f // D == h).astype(jnp.float32)


def _partial_body(q_ref, k_ref, v_ref, o_ref, m_ref, l_ref):
    b = pl.program_id(0)
    qcol = jnp.reshape(q_ref[pl.ds(b, 1), :], (F, 1))
    w = qcol * _mask_fh()
    k2 = k_ref[0]
    s = jax.lax.dot_general(
        k2, w, (((1,), (0,)), ((), ())),
        preferred_element_type=jnp.float32,
    ) * SCALE
    m = jnp.max(s, axis=0, keepdims=True)
    p = jnp.exp(s - m)
    l = jnp.sum(p, axis=0, keepdims=True)
    pe = jax.lax.dot_general(
        p, _expand_hf(), (((1,), (0,)), ((), ())),
        preferred_element_type=jnp.float32,
    )
    o_ref[pl.ds(b, 1), :] = jnp.sum(pe * v_ref[0], axis=0, keepdims=True)
    m_ref[pl.ds(b, 1), :] = m
    l_ref[pl.ds(b, 1), :] = l


def _partials(q2, k2, v2):
    return pl.pallas_call(
        _partial_body,
        grid=(B,),
        in_specs=[
            pl.BlockSpec((B, F), lambda b: (0, 0)),
            pl.BlockSpec((1, SEQ, F), lambda b: (b, 0, 0)),
            pl.BlockSpec((1, SEQ, F), lambda b: (b, 0, 0)),
        ],
        out_specs=[
            pl.BlockSpec((B, F), lambda b: (0, 0)),
            pl.BlockSpec((B, H), lambda b: (0, 0)),
            pl.BlockSpec((B, H), lambda b: (0, 0)),
        ],
        out_shape=[
            jax.ShapeDtypeStruct((B, F), jnp.float32),
            jax.ShapeDtypeStruct((B, H), jnp.float32),
            jax.ShapeDtypeStruct((B, H), jnp.float32),
        ],
    )(q2, k2, v2)


def _combine_body(o_ref, m_ref, l_ref, out_ref,
                  co, cm, cl, so, sm, sl, ro, rm, rl):
    my = lax.axis_index("i")

    bar = pltpu.get_barrier_semaphore()
    for k in (1, 2, 3):
        pl.semaphore_signal(
            bar, inc=1,
            device_id=((my + k) % N_DEV,),
            device_id_type=pl.DeviceIdType.MESH,
        )
    pl.semaphore_wait(bar, 3)

    co[0] = o_ref[...]
    cm[0] = m_ref[...]
    cl[0] = l_ref[...]

    rdmas = []
    for k in (1, 2, 3):
        tgt = (my + k) % N_DEV
        slot = N_DEV - k
        for comm, ssem, rsem in ((co, so, ro), (cm, sm, rm), (cl, sl, rl)):
            r = pltpu.make_async_remote_copy(
                src_ref=comm.at[0],
                dst_ref=comm.at[slot],
                send_sem=ssem.at[k],
                recv_sem=rsem.at[slot],
                device_id=(tgt,),
                device_id_type=pl.DeviceIdType.MESH,
            )
            r.start()
            rdmas.append(r)
    for r in rdmas:
        r.wait()

    e = _expand_hf()
    ms = [cm[i] for i in range(N_DEV)]
    mg = jnp.maximum(jnp.maximum(ms[0], ms[1]), jnp.maximum(ms[2], ms[3]))
    lg = jnp.zeros((B, H), jnp.float32)
    acc = jnp.zeros((B, F), jnp.float32)
    for i in range(N_DEV):
        wi = jnp.exp(ms[i] - mg)
        lg = lg + wi * cl[i]
        wf = jax.lax.dot_general(
            wi, e, (((1,), (0,)), ((), ())),
            preferred_element_type=jnp.float32,
        )
        acc = acc + wf * co[i]
    lgf = jax.lax.dot_general(
        lg, e, (((1,), (0,)), ((), ())),
        preferred_element_type=jnp.float32,
    )
    out_ref[...] = acc / lgf


def _combine(o_part, m_part, l_part):
    return pl.pallas_call(
        _combine_body,
        in_specs=[
            pl.BlockSpec(memory_space=pltpu.VMEM),
            pl.BlockSpec(memory_space=pltpu.VMEM),
            pl.BlockSpec(memory_space=pltpu.VMEM),
        ],
        out_specs=pl.BlockSpec(memory_space=pltpu.VMEM),
        out_shape=jax.ShapeDtypeStruct((B, F), jnp.float32),
        scratch_shapes=[
            pltpu.VMEM((N_DEV, B, F), jnp.float32),
            pltpu.VMEM((N_DEV, B, H), jnp.float32),
            pltpu.VMEM((N_DEV, B, H), jnp.float32),
            pltpu.SemaphoreType.DMA((N_DEV,)),
            pltpu.SemaphoreType.DMA((N_DEV,)),
            pltpu.SemaphoreType.DMA((N_DEV,)),
            pltpu.SemaphoreType.DMA((N_DEV,)),
            pltpu.SemaphoreType.DMA((N_DEV,)),
            pltpu.SemaphoreType.DMA((N_DEV,)),
        ],
        compiler_params=pltpu.CompilerParams(collective_id=0),
    )(o_part, m_part, l_part)


def kernel(Q, K, V):
    q2 = Q.reshape(B, F)
    k2 = K.reshape(B, SEQ, F)
    v2 = V.reshape(B, SEQ, F)
    o_part, m_part, l_part = _partials(q2, k2, v2)
    out = _combine(o_part, m_part, l_part)
    return out.reshape(B, 1, H, D)


# device time: 49011 ns/iter; 3.8275x vs baseline; 3.8275x over previous
import jax
import jax.numpy as jnp
from jax import lax
from jax.experimental import pallas as pl
from jax.experimental.pallas import tpu as pltpu

N_DEV = 4
B = 16
SEQ = 1024
H = 16
D = 64
F = H * D
SCALE = D ** -0.5


def _expand_hf():
    h = lax.broadcasted_iota(jnp.int32, (H, F), 0)
    f = lax.broadcasted_iota(jnp.int32, (H, F), 1)
    return (f // D == h).astype(jnp.float32)


def _partial_body(q_ref, kt_ref, vt_ref, o_ref, m_ref, l_ref,
                  s_scr, row_scr, ml_scr):
    b = pl.program_id(0)
    qrow = q_ref[pl.ds(b, 1), :]
    for h in range(H):
        qh = qrow[0:1, h * D:(h + 1) * D]
        s_scr[pl.ds(h, 1), :] = jax.lax.dot_general(
            qh, kt_ref[0, h], (((1,), (0,)), ((), ())),
            preferred_element_type=jnp.float32,
        )
    s = s_scr[...] * SCALE
    m = jnp.max(s, axis=1, keepdims=True)
    p = jnp.exp(s - m)
    l = jnp.sum(p, axis=1, keepdims=True)
    for h in range(H):
        row_scr[0:1, pl.ds(h * D, D)] = jax.lax.dot_general(
            p[h:h + 1, :], vt_ref[0, h], (((1,), (1,)), ((), ())),
            preferred_element_type=jnp.float32,
        )
        ml_scr[0:1, h:h + 1] = m[h:h + 1, :]
        ml_scr[1:2, h:h + 1] = l[h:h + 1, :]
    o_ref[pl.ds(b, 1), :] = row_scr[...]
    m_ref[pl.ds(b, 1), :] = ml_scr[0:1, :]
    l_ref[pl.ds(b, 1), :] = ml_scr[1:2, :]


def _partials(q2, kt, vt):
    return pl.pallas_call(
        _partial_body,
        grid=(B,),
        in_specs=[
            pl.BlockSpec((B, F), lambda b: (0, 0)),
            pl.BlockSpec((1, H, D, SEQ), lambda b: (b, 0, 0, 0)),
            pl.BlockSpec((1, H, D, SEQ), lambda b: (b, 0, 0, 0)),
        ],
        out_specs=[
            pl.BlockSpec((B, F), lambda b: (0, 0)),
            pl.BlockSpec((B, H), lambda b: (0, 0)),
            pl.BlockSpec((B, H), lambda b: (0, 0)),
        ],
        out_shape=[
            jax.ShapeDtypeStruct((B, F), jnp.float32),
            jax.ShapeDtypeStruct((B, H), jnp.float32),
            jax.ShapeDtypeStruct((B, H), jnp.float32),
        ],
        scratch_shapes=[
            pltpu.VMEM((H, SEQ), jnp.float32),
            pltpu.VMEM((1, F), jnp.float32),
            pltpu.VMEM((2, H), jnp.float32),
        ],
    )(q2, kt, vt)


def _combine_body(o_ref, m_ref, l_ref, out_ref,
                  co, cm, cl, so, sm, sl, ro, rm, rl):
    my = lax.axis_index("i")

    bar = pltpu.get_barrier_semaphore()
    for k in (1, 2, 3):
        pl.semaphore_signal(
            bar, inc=1,
            device_id=((my + k) % N_DEV,),
            device_id_type=pl.DeviceIdType.MESH,
        )
    pl.semaphore_wait(bar, 3)

    co[0] = o_ref[...]
    cm[0] = m_ref[...]
    cl[0] = l_ref[...]

    rdmas = []
    for k in (1, 2, 3):
        tgt = (my + k) % N_DEV
        slot = N_DEV - k
        for comm, ssem, rsem in ((co, so, ro), (cm, sm, rm), (cl, sl, rl)):
            r = pltpu.make_async_remote_copy(
                src_ref=comm.at[0],
                dst_ref=comm.at[slot],
                send_sem=ssem.at[k],
                recv_sem=rsem.at[slot],
                device_id=(tgt,),
                device_id_type=pl.DeviceIdType.MESH,
            )
            r.start()
            rdmas.append(r)
    for r in rdmas:
        r.wait()

    e = _expand_hf()
    ms = [cm[i] for i in range(N_DEV)]
    mg = jnp.maximum(jnp.maximum(ms[0], ms[1]), jnp.maximum(ms[2], ms[3]))
    lg = jnp.zeros((B, H), jnp.float32)
    acc = jnp.zeros((B, F), jnp.float32)
    for i in range(N_DEV):
        wi = jnp.exp(ms[i] - mg)
        lg = lg + wi * cl[i]
        wf = jax.lax.dot_general(
            wi, e, (((1,), (0,)), ((), ())),
            preferred_element_type=jnp.float32,
        )
        acc = acc + wf * co[i]
    lgf = jax.lax.dot_general(
        lg, e, (((1,), (0,)), ((), ())),
        preferred_element_type=jnp.float32,
    )
    out_ref[...] = acc / lgf


def _combine(o_part, m_part, l_part):
    return pl.pallas_call(
        _combine_body,
        in_specs=[
            pl.BlockSpec(memory_space=pltpu.VMEM),
            pl.BlockSpec(memory_space=pltpu.VMEM),
            pl.BlockSpec(memory_space=pltpu.VMEM),
        ],
        out_specs=pl.BlockSpec(memory_space=pltpu.VMEM),
        out_shape=jax.ShapeDtypeStruct((B, F), jnp.float32),
        scratch_shapes=[
            pltpu.VMEM((N_DEV, B, F), jnp.float32),
            pltpu.VMEM((N_DEV, B, H), jnp.float32),
            pltpu.VMEM((N_DEV, B, H), jnp.float32),
            pltpu.SemaphoreType.DMA((N_DEV,)),
            pltpu.SemaphoreType.DMA((N_DEV,)),
            pltpu.SemaphoreType.DMA((N_DEV,)),
            pltpu.SemaphoreType.DMA((N_DEV,)),
            pltpu.SemaphoreType.DMA((N_DEV,)),
            pltpu.SemaphoreType.DMA((N_DEV,)),
        ],
        compiler_params=pltpu.CompilerParams(collective_id=0),
    )(o_part, m_part, l_part)


def kernel(Q, K, V):
    q2 = Q.reshape(B, F)
    kt = K.transpose(0, 2, 3, 1)
    vt = V.transpose(0, 2, 3, 1)
    o_part, m_part, l_part = _partials(q2, kt, vt)
    out = _combine(o_part, m_part, l_part)
    return out.reshape(B, 1, H, D)


# device time: 48474 ns/iter; 3.8699x vs baseline; 1.0111x over previous
import jax
import jax.numpy as jnp
from jax import lax
from jax.experimental import pallas as pl
from jax.experimental.pallas import tpu as pltpu

N_DEV = 4
B = 16
SEQ = 1024
H = 16
D = 64
F = H * D
PF = 1280
MO = 1024
LO = 1152
SCALE = D ** -0.5


def _expand_hf():
    h = lax.broadcasted_iota(jnp.int32, (H, F), 0)
    f = lax.broadcasted_iota(jnp.int32, (H, F), 1)
    return (f // D == h).astype(jnp.float32)


def _body(q_ref, kt_ref, vt_ref, out_ref,
          po, c1, c2, c3, s_scr, row_scr, ss, rs):
    b = pl.program_id(0)
    my = lax.axis_index("i")
    peer_bufs = {1: c3, 2: c2, 3: c1}

    @pl.when(b == 0)
    def _():
        bar = pltpu.get_barrier_semaphore()
        for k in (1, 2, 3):
            pl.semaphore_signal(
                bar, inc=1,
                device_id=((my + k) % N_DEV,),
                device_id_type=pl.DeviceIdType.MESH,
            )
        pl.semaphore_wait(bar, 3)

    qrow = q_ref[pl.ds(b, 1), :]
    for h in range(H):
        qh = qrow[0:1, h * D:(h + 1) * D]
        s_scr[pl.ds(h, 1), :] = jax.lax.dot_general(
            qh, kt_ref[0, h], (((1,), (0,)), ((), ())),
            preferred_element_type=jnp.float32,
        )
    s = s_scr[...] * SCALE
    m = jnp.max(s, axis=1, keepdims=True)
    p = jnp.exp(s - m)
    l = jnp.sum(p, axis=1, keepdims=True)
    for h in range(H):
        row_scr[0:1, pl.ds(h * D, D)] = jax.lax.dot_general(
            p[h:h + 1, :], vt_ref[0, h], (((1,), (1,)), ((), ())),
            preferred_element_type=jnp.float32,
        )
        row_scr[0:1, MO + h:MO + h + 1] = m[h:h + 1, :]
        row_scr[0:1, LO + h:LO + h + 1] = l[h:h + 1, :]
    po[pl.ds(b, 1), :] = row_scr[...]

    for k in (1, 2, 3):
        r = pltpu.make_async_remote_copy(
            src_ref=po.at[pl.ds(b, 1)],
            dst_ref=peer_bufs[4 - k].at[pl.ds(b, 1)],
            send_sem=ss.at[k - 1],
            recv_sem=rs.at[k - 1],
            device_id=((my + k) % N_DEV,),
            device_id_type=pl.DeviceIdType.MESH,
        )
        r.start()

    @pl.when(b == B - 1)
    def _():
        for k in (1, 2, 3):
            for _i in range(B):
                w = pltpu.make_async_remote_copy(
                    src_ref=po.at[pl.ds(0, 1)],
                    dst_ref=peer_bufs[4 - k].at[pl.ds(0, 1)],
                    send_sem=ss.at[k - 1],
                    recv_sem=rs.at[k - 1],
                    device_id=((my + k) % N_DEV,),
                    device_id_type=pl.DeviceIdType.MESH,
                )
                w.wait_send()
                w.wait_recv()

        e = _expand_hf()
        parts = [po, c1, c2, c3]
        ms = [c[:, MO:MO + H] for c in parts]
        mg = jnp.maximum(jnp.maximum(ms[0], ms[1]),
                         jnp.maximum(ms[2], ms[3]))
        lg = jnp.zeros((B, H), jnp.float32)
        acc = jnp.zeros((B, F), jnp.float32)
        for i in range(N_DEV):
            wi = jnp.exp(ms[i] - mg)
            lg = lg + wi * parts[i][:, LO:LO + H]
            wf = jax.lax.dot_general(
                wi, e, (((1,), (0,)), ((), ())),
                preferred_element_type=jnp.float32,
            )
            acc = acc + wf * parts[i][:, 0:F]
        lgf = jax.lax.dot_general(
            lg, e, (((1,), (0,)), ((), ())),
            preferred_element_type=jnp.float32,
        )
        out_ref[...] = acc / lgf


def kernel(Q, K, V):
    q2 = Q.reshape(B, F)
    kt = K.transpose(0, 2, 3, 1)
    vt = V.transpose(0, 2, 3, 1)
    out = pl.pallas_call(
        _body,
        grid=(B,),
        in_specs=[
            pl.BlockSpec((B, F), lambda b: (0, 0)),
            pl.BlockSpec((1, H, D, SEQ), lambda b: (b, 0, 0, 0)),
            pl.BlockSpec((1, H, D, SEQ), lambda b: (b, 0, 0, 0)),
        ],
        out_specs=pl.BlockSpec((B, F), lambda b: (0, 0)),
        out_shape=jax.ShapeDtypeStruct((B, F), jnp.float32),
        scratch_shapes=[
            pltpu.VMEM((B, PF), jnp.float32),
            pltpu.VMEM((B, PF), jnp.float32),
            pltpu.VMEM((B, PF), jnp.float32),
            pltpu.VMEM((B, PF), jnp.float32),
            pltpu.VMEM((H, SEQ), jnp.float32),
            pltpu.VMEM((1, PF), jnp.float32),
            pltpu.SemaphoreType.DMA((3,)),
            pltpu.SemaphoreType.DMA((3,)),
        ],
        compiler_params=pltpu.CompilerParams(collective_id=0),
    )(q2, kt, vt)
    return out.reshape(B, 1, H, D)
